# baseline (speedup 1.0000x reference)
import functools

import jax
import jax.numpy as jnp
from jax import lax
from jax.experimental import pallas as pl
from jax.experimental.pallas import tpu as pltpu
from jax.experimental.pallas import tpu_sc as plsc

B = 16384
D = 32
L = 16
NC = 2
NS = 16
HALF = B // NC
CHUNK = HALF // NS


def _sc_body(user_hbm, item_hbm, ut_hbm, it_hbm, out_hbm,
             uidx_v, iidx_v, gu0, gu1, gi0, gi1, prod_v, tmp_v, acc_v,
             shared_sp, s0, s1, s2, s3):
    c = lax.axis_index("c")
    s = lax.axis_index("s")
    half_base = c * HALF

    pltpu.sync_copy(user_hbm.at[pl.ds(half_base, HALF)], uidx_v)
    pltpu.sync_copy(item_hbm.at[pl.ds(half_base, HALF)], iidx_v)
    d0 = 2 * s
    d1 = 2 * s + 1
    cu0 = pltpu.async_copy(ut_hbm.at[d0].at[uidx_v], gu0, s0)
    cu1 = pltpu.async_copy(ut_hbm.at[d1].at[uidx_v], gu1, s1)
    ci0 = pltpu.async_copy(it_hbm.at[d0].at[iidx_v], gi0, s2)
    ci1 = pltpu.async_copy(it_hbm.at[d1].at[iidx_v], gi1, s3)
    cu0.wait()
    cu1.wait()
    ci0.wait()
    ci1.wait()

    def pbody(k, carry):
        off = k * L
        p = (gu0[pl.ds(off, L)] * gi0[pl.ds(off, L)]
             + gu1[pl.ds(off, L)] * gi1[pl.ds(off, L)])
        prod_v[pl.ds(off, L)] = p
        return carry

    lax.fori_loop(0, HALF // L, pbody, 0)

    pltpu.sync_copy(prod_v, shared_sp.at[s])
    plsc.subcore_barrier()

    chunk_off = s * CHUNK

    def accum(t, carry):
        pltpu.sync_copy(shared_sp.at[t, pl.ds(chunk_off, CHUNK)], tmp_v)

        def addk(k, c2):
            off = k * L
            acc_v[pl.ds(off, L)] = acc_v[pl.ds(off, L)] + tmp_v[pl.ds(off, L)]
            return c2

        lax.fori_loop(0, CHUNK // L, addk, 0)
        return carry

    pltpu.sync_copy(shared_sp.at[0, pl.ds(chunk_off, CHUNK)], acc_v)
    lax.fori_loop(1, NS, accum, 0)
    pltpu.sync_copy(acc_v, out_hbm.at[pl.ds(half_base + chunk_off, CHUNK)])


@jax.jit
def kernel(user, item, user_emb, item_emb):
    mesh = plsc.VectorSubcoreMesh(
        core_axis_name="c", subcore_axis_name="s",
        num_cores=NC, num_subcores=NS,
    )
    run = pl.kernel(
        _sc_body,
        out_type=jax.ShapeDtypeStruct((B,), jnp.float32),
        mesh=mesh,
        scratch_types=[
            pltpu.VMEM((HALF,), jnp.int32),
            pltpu.VMEM((HALF,), jnp.int32),
            pltpu.VMEM((HALF,), jnp.float32),
            pltpu.VMEM((HALF,), jnp.float32),
            pltpu.VMEM((HALF,), jnp.float32),
            pltpu.VMEM((HALF,), jnp.float32),
            pltpu.VMEM((HALF,), jnp.float32),
            pltpu.VMEM((CHUNK,), jnp.float32),
            pltpu.VMEM((CHUNK,), jnp.float32),
            pltpu.VMEM_SHARED((NS, HALF), jnp.float32),
            pltpu.SemaphoreType.DMA,
            pltpu.SemaphoreType.DMA,
            pltpu.SemaphoreType.DMA,
            pltpu.SemaphoreType.DMA,
        ],
        compiler_params=pltpu.CompilerParams(
            needs_layout_passes=False, use_tc_tiling_on_sc=False,
        ),
    )
    return run(user, item, user_emb.T, item_emb.T)


# speedup vs baseline: 5.6503x; 5.6503x over previous
import functools

import jax
import jax.numpy as jnp
from jax import lax
from jax.experimental import pallas as pl
from jax.experimental.pallas import tpu as pltpu
from jax.experimental.pallas import tpu_sc as plsc

B = 16384
D = 32
L = 16
NC = 2
NS = 16
NW = NC * NS
BPW = B // NW
GROUPS = BPW // L


def _sc_body(user_hbm, item_hbm, uemb_hbm, iemb_hbm, out_hbm,
             uidx_v, iidx_v, urows_v, irows_v, out_v, sem_u, sem_i):
    wid = lax.axis_index("s") * NC + lax.axis_index("c")
    base = wid * BPW

    pltpu.sync_copy(user_hbm.at[pl.ds(base, BPW)], uidx_v)
    pltpu.sync_copy(item_hbm.at[pl.ds(base, BPW)], iidx_v)
    cu = pltpu.async_copy(uemb_hbm.at[uidx_v], urows_v, sem_u)
    ci = pltpu.async_copy(iemb_hbm.at[iidx_v], irows_v, sem_i)
    cu.wait()
    ci.wait()

    lane = lax.iota(jnp.int32, L)

    def group(g, carry):
        row = g * L + lane
        acc = jnp.zeros((L,), jnp.float32)
        for d in range(D):
            col = jnp.full((L,), d, jnp.int32)
            uu = plsc.load_gather(urows_v, [row, col])
            vv = plsc.load_gather(irows_v, [row, col])
            acc = acc + uu * vv
        out_v[pl.ds(g * L, L)] = acc
        return carry

    lax.fori_loop(0, GROUPS, group, 0)
    pltpu.sync_copy(out_v, out_hbm.at[pl.ds(base, BPW)])


@jax.jit
def kernel(user, item, user_emb, item_emb):
    mesh = plsc.VectorSubcoreMesh(
        core_axis_name="c", subcore_axis_name="s",
        num_cores=NC, num_subcores=NS,
    )
    run = pl.kernel(
        _sc_body,
        out_type=jax.ShapeDtypeStruct((B,), jnp.float32),
        mesh=mesh,
        scratch_types=[
            pltpu.VMEM((BPW,), jnp.int32),
            pltpu.VMEM((BPW,), jnp.int32),
            pltpu.VMEM((BPW, D), jnp.float32),
            pltpu.VMEM((BPW, D), jnp.float32),
            pltpu.VMEM((BPW,), jnp.float32),
            pltpu.SemaphoreType.DMA,
            pltpu.SemaphoreType.DMA,
        ],
        compiler_params=pltpu.CompilerParams(
            needs_layout_passes=False, use_tc_tiling_on_sc=False,
        ),
    )
    return run(user, item, user_emb, item_emb)


# speedup vs baseline: 5.6581x; 1.0014x over previous
import jax
import jax.numpy as jnp
from jax import lax
from jax.experimental import pallas as pl
from jax.experimental.pallas import tpu as pltpu
from jax.experimental.pallas import tpu_sc as plsc

B = 16384
D = 32
L = 16
NC = 2
NS = 16
NW = NC * NS
BPW = B // NW
GROUPS = BPW // L


def _sc_body(user_hbm, item_hbm, uemb_hbm, iemb_hbm, out_hbm,
             uidx_v, iidx_v, urows_v, irows_v, out_v, sem_u, sem_i):
    wid = lax.axis_index("s") * NC + lax.axis_index("c")
    base = wid * BPW

    pltpu.sync_copy(user_hbm.at[pl.ds(base, BPW)], uidx_v)
    pltpu.sync_copy(item_hbm.at[pl.ds(base, BPW)], iidx_v)
    cu = pltpu.async_copy(uemb_hbm.at[uidx_v], urows_v, sem_u)
    ci = pltpu.async_copy(iemb_hbm.at[iidx_v], irows_v, sem_i)
    cu.wait()
    ci.wait()

    lane = lax.iota(jnp.int32, L)

    def group(g, carry):
        row = g * L + lane
        acc = jnp.zeros((L,), jnp.float32)
        for d in range(D):
            col = jnp.full((L,), d, jnp.int32)
            uu = plsc.load_gather(urows_v, [row, col])
            vv = plsc.load_gather(irows_v, [row, col])
            acc = acc + uu * vv
        out_v[pl.ds(g * L, L)] = acc
        return carry

    lax.fori_loop(0, GROUPS, group, 0)
    pltpu.sync_copy(out_v, out_hbm.at[pl.ds(base, BPW)])


@jax.jit
def kernel(user, item, user_emb, item_emb):
    mesh = plsc.VectorSubcoreMesh(
        core_axis_name="c", subcore_axis_name="s",
        num_cores=NC, num_subcores=NS,
    )
    run = pl.kernel(
        _sc_body,
        out_type=jax.ShapeDtypeStruct((B,), jnp.float32),
        mesh=mesh,
        scratch_types=[
            pltpu.VMEM((BPW,), jnp.int32),
            pltpu.VMEM((BPW,), jnp.int32),
            pltpu.VMEM((BPW, D), jnp.float32),
            pltpu.VMEM((BPW, D), jnp.float32),
            pltpu.VMEM((BPW,), jnp.float32),
            pltpu.SemaphoreType.DMA,
            pltpu.SemaphoreType.DMA,
        ],
        compiler_params=pltpu.CompilerParams(
            needs_layout_passes=False, use_tc_tiling_on_sc=False,
        ),
    )
    return run(user, item, user_emb, item_emb)
